# baseline (speedup 1.0000x reference)
import functools

import jax
import jax.numpy as jnp
from jax import lax
from jax.experimental import pallas as pl
from jax.experimental.pallas import tpu as pltpu
from jax.experimental.pallas import tpu_sc as plsc

_LANES = 16
_NBUF = 2


@functools.cache
def _build(n_nodes: int, n_edges: int, d_feat: int):
    info = plsc.get_sparse_core_info()
    nc, ns = info.num_cores, info.num_subcores
    nw = nc * ns
    assert n_edges % nw == 0
    per_w = n_edges // nw
    rows_per_s = (n_nodes // ns) // 8 * 8
    rows_rem = n_nodes - ns * rows_per_s
    assert rows_rem % 8 == 0
    chunk = 40
    assert per_w % chunk == 0
    n_chunks = per_w // chunk
    assert n_chunks % _NBUF == 0
    n_vec = d_feat // _LANES

    mesh = plsc.VectorSubcoreMesh(core_axis_name="c", subcore_axis_name="s")

    @functools.partial(
        pl.kernel,
        mesh=mesh,
        out_type=jax.ShapeDtypeStruct((n_edges, d_feat), jnp.float32),
        scratch_types=[
            pltpu.VMEM_SHARED((n_nodes, d_feat), jnp.float32),
            pltpu.VMEM((per_w,), jnp.int32),
            pltpu.VMEM((per_w,), jnp.int32),
            pltpu.VMEM((_NBUF, chunk, d_feat), jnp.float32),
            pltpu.VMEM((_NBUF, chunk, d_feat), jnp.float32),
            pltpu.VMEM((_NBUF, chunk, d_feat), jnp.float32),
            pltpu.SemaphoreType.DMA((_NBUF,)),
            pltpu.SemaphoreType.DMA((_NBUF,)),
        ],
    )
    def edge_diff(x_hbm, src_hbm, dst_hbm, out_hbm,
                  xs_sh, src_v, dst_v, a_v, b_v, o_v, sem_g, sem_o):
        sid = lax.axis_index("s")
        wid = sid * nc + lax.axis_index("c")
        base = wid * per_w

        row0 = sid * rows_per_s
        pltpu.sync_copy(x_hbm.at[pl.ds(row0, rows_per_s)],
                        xs_sh.at[pl.ds(row0, rows_per_s)])
        if rows_rem:
            @pl.when(sid == ns - 1)
            def _stage_rem():
                pltpu.sync_copy(
                    x_hbm.at[pl.ds(ns * rows_per_s, rows_rem)],
                    xs_sh.at[pl.ds(ns * rows_per_s, rows_rem)])
        pltpu.sync_copy(src_hbm.at[pl.ds(base, per_w)], src_v)
        pltpu.sync_copy(dst_hbm.at[pl.ds(base, per_w)], dst_v)
        plsc.subcore_barrier()

        def start_gathers(c, b):
            pltpu.async_copy(
                xs_sh.at[src_v.at[pl.ds(c * chunk, chunk)]], a_v.at[b],
                sem_g.at[b])
            pltpu.async_copy(
                xs_sh.at[dst_v.at[pl.ds(c * chunk, chunk)]], b_v.at[b],
                sem_g.at[b])

        for b in range(_NBUF):
            start_gathers(b, b)

        idx0 = src_v.at[pl.ds(0, chunk)]

        def do_group(g, carry):
            for b in range(_NBUF):
                c = g * _NBUF + b
                off = base + c * chunk
                pltpu.make_async_copy(
                    xs_sh.at[idx0], a_v.at[b], sem_g.at[b]).wait()
                pltpu.make_async_copy(
                    xs_sh.at[idx0], b_v.at[b], sem_g.at[b]).wait()

                @pl.when(g > 0)
                def _wait_out():
                    pltpu.make_async_copy(
                        o_v.at[b], out_hbm.at[pl.ds(off, chunk)],
                        sem_o.at[b]).wait()

                @plsc.parallel_loop(0, chunk, unroll=4)
                def _sub(r):
                    for v in range(n_vec):
                        sl = pl.ds(v * _LANES, _LANES)
                        o_v[b, r, sl] = a_v[b, r, sl] - b_v[b, r, sl]

                pltpu.async_copy(
                    o_v.at[b], out_hbm.at[pl.ds(off, chunk)], sem_o.at[b])

                @pl.when(c + _NBUF < n_chunks)
                def _prefetch():
                    start_gathers(c + _NBUF, b)
            return carry

        lax.fori_loop(0, n_chunks // _NBUF, do_group, 0)

        for b in range(_NBUF):
            off = base + (n_chunks - _NBUF + b) * chunk
            pltpu.make_async_copy(
                o_v.at[b], out_hbm.at[pl.ds(off, chunk)], sem_o.at[b]).wait()

    return edge_diff


def kernel(x, edge_index):
    ei = edge_index.astype(jnp.int32)
    fn = _build(x.shape[0], ei.shape[1], x.shape[1])
    return fn(x, ei[0], ei[1])
